# baseline (speedup 1.0000x reference)
import functools

import jax
import jax.numpy as jnp
from jax.experimental import pallas as pl

_W = 512


def _quantize_one(res, row):
    a = jnp.abs(res)
    neg = (res < 0.0)
    negi = neg.astype(jnp.int32)

    m1 = jnp.max(a, axis=0)
    i1 = jnp.min(jnp.where(a == m1[None, :], row, 8), axis=0)
    mask1 = row == i1[None, :]
    a2 = jnp.where(mask1, -1.0, a)
    m2 = jnp.max(a2, axis=0)
    i2 = jnp.min(jnp.where(a2 == m2[None, :], row, 8), axis=0)
    score1 = m1 + m2

    total = jnp.sum(a, axis=0)
    mn = jnp.min(a, axis=0)
    tied = a == mn[None, :]
    tiedneg = tied & neg
    has_tn = jnp.max(jnp.where(tiedneg, 1, 0), axis=0) == 1
    k_neg = jnp.max(jnp.where(tiedneg, row, -1), axis=0)
    k_pos = jnp.min(jnp.where(tied, row, 8), axis=0)
    imn = jnp.where(has_tn, k_neg, k_pos)
    parity_odd = (jnp.sum(negi, axis=0) & 1) == 1
    score2 = 0.5 * total - jnp.where(parity_odd, mn, 0.0)

    use1 = score1 >= score2

    lo = jnp.minimum(i1, i2)
    hi = jnp.maximum(i1, i2)
    pair = lo * (15 - lo) // 2 + (hi - lo - 1)
    s_lo = jnp.sum(jnp.where(row == lo[None, :], negi, 0), axis=0)
    s_hi = jnp.sum(jnp.where(row == hi[None, :], negi, 0), axis=0)
    index1 = 4 * pair + 2 * s_lo + s_hi

    flip = parity_odd[None, :] & (row == imn[None, :])
    bits = jnp.where(flip, 1 - negi, negi)
    b = jnp.sum(bits << row, axis=0)
    index2 = 112 + (b >> 1)

    idx = jnp.where(use1, index1, index2)

    sgnval = jnp.where(neg, -1.0, 1.0)
    y_t1 = jnp.where(mask1 | (row == i2[None, :]), sgnval, 0.0)
    y_t2 = jnp.where(bits == 1, -0.5, 0.5)
    y = jnp.where(use1[None, :], y_t1, y_t2)
    return y, idx


def _bf16(v):
    return v.astype(jnp.bfloat16).astype(jnp.float32)


def _body(x_ref, q_ref, i1_ref, i2_ref, err_ref):
    L = x_ref.shape[1]
    row = jax.lax.broadcasted_iota(jnp.int32, (8, _W), 0)
    err_acc = jnp.zeros((1, 1), jnp.float32)

    for c in range(L // _W):
        sl = pl.ds(c * _W, _W)
        xt = x_ref[:, sl]
        y1, idx1 = _quantize_one(_bf16(xt), row)
        r1 = xt - y1
        y2, idx2 = _quantize_one(_bf16(r1), row)
        q = y1 + y2
        q_ref[:, sl] = q
        i1_ref[0, 0, sl] = idx1
        i2_ref[0, 0, sl] = idx2
        resid = xt - q
        err_acc = err_acc + jnp.sum(resid * resid).reshape(1, 1)

    @pl.when(pl.program_id(0) == 0)
    def _init():
        err_ref[...] = err_acc

    @pl.when(pl.program_id(0) != 0)
    def _acc():
        err_ref[...] += err_acc


@functools.partial(jax.jit, static_argnames=("interpret",))
def kernel(x, roots, interpret=False):
    del roots
    orig_shape = x.shape
    n = x.shape[0] * x.shape[1]
    L = 8192
    g = n // L

    eye8 = jnp.eye(8, dtype=jnp.float32)
    xt = jax.lax.dot_general(
        eye8, x.reshape(n, 8), (((1,), (1,)), ((), ())),
        precision=jax.lax.Precision.HIGHEST,
    )

    q_t, idx1, idx2, err_sum = pl.pallas_call(
        _body,
        grid=(g,),
        in_specs=[pl.BlockSpec((8, L), lambda i: (0, i))],
        out_specs=(
            pl.BlockSpec((8, L), lambda i: (0, i)),
            pl.BlockSpec((1, 1, L), lambda i: (i, 0, 0)),
            pl.BlockSpec((1, 1, L), lambda i: (i, 0, 0)),
            pl.BlockSpec((1, 1), lambda i: (0, 0)),
        ),
        out_shape=(
            jax.ShapeDtypeStruct((8, n), jnp.float32),
            jax.ShapeDtypeStruct((g, 1, L), jnp.int32),
            jax.ShapeDtypeStruct((g, 1, L), jnp.int32),
            jax.ShapeDtypeStruct((1, 1), jnp.float32),
        ),
        interpret=interpret,
    )(xt)

    quantized = jax.lax.dot_general(
        q_t, eye8, (((0,), (0,)), ((), ())),
        precision=jax.lax.Precision.HIGHEST,
    ).reshape(orig_shape)
    i1 = idx1.reshape(orig_shape[:-1])
    i2 = idx2.reshape(orig_shape[:-1])
    err = err_sum[0, 0] / jnp.float32(n * 8)
    return (quantized, i1, i2, err)


# speedup vs baseline: 1.5017x; 1.5017x over previous
import functools

import jax
import jax.numpy as jnp
import numpy as np
from jax.experimental import pallas as pl
from jax.experimental.pallas import tpu as pltpu

_ROWS = 64
_SUB = 16


def _perm_np():
    p = np.zeros((1024, 1024), np.float32)
    l = np.arange(1024)
    p[l, (l % 8) * 128 + l // 8] = 1.0
    return p


def _bf16(v):
    return v.astype(jnp.bfloat16).astype(jnp.float32)


def _quantize_list(rs):
    a = [jnp.abs(r) for r in rs]
    neg = [r < 0.0 for r in rs]
    negi = [n.astype(jnp.int32) for n in neg]

    m1 = a[0]
    i1 = jnp.zeros_like(negi[0])
    for d in range(1, 8):
        b = a[d] > m1
        m1 = jnp.where(b, a[d], m1)
        i1 = jnp.where(b, d, i1)
    m2 = jnp.full_like(m1, -1.0)
    i2 = jnp.zeros_like(i1)
    for d in range(8):
        ad = jnp.where(i1 == d, -1.0, a[d])
        b = ad > m2
        m2 = jnp.where(b, ad, m2)
        i2 = jnp.where(b, d, i2)
    score1 = m1 + m2

    total = a[0]
    mn = a[0]
    for d in range(1, 8):
        total = total + a[d]
        mn = jnp.minimum(mn, a[d])

    has_tn = (a[0] == mn) & neg[0]
    k_neg = jnp.zeros_like(i1)
    for d in range(8):
        tn = (a[d] == mn) & neg[d]
        k_neg = jnp.where(tn, d, k_neg)
        if d > 0:
            has_tn = has_tn | tn
    k_pos = jnp.zeros_like(i1)
    for d in range(7, -1, -1):
        k_pos = jnp.where(a[d] == mn, d, k_pos)
    imn = jnp.where(has_tn, k_neg, k_pos)

    par = negi[0]
    for d in range(1, 8):
        par = par ^ negi[d]
    parity_odd = par == 1
    score2 = 0.5 * total - jnp.where(parity_odd, mn, 0.0)

    use1 = score1 >= score2

    lo = jnp.minimum(i1, i2)
    hi = jnp.maximum(i1, i2)
    pair = (lo * (15 - lo) >> 1) + (hi - lo - 1)
    s_lo = jnp.zeros_like(i1)
    s_hi = jnp.zeros_like(i1)
    for d in range(8):
        s_lo = jnp.where(lo == d, negi[d], s_lo)
        s_hi = jnp.where(hi == d, negi[d], s_hi)
    index1 = 4 * pair + 2 * s_lo + s_hi

    bits = []
    b = jnp.zeros_like(i1)
    for d in range(8):
        fd = parity_odd & (imn == d)
        bd = jnp.where(fd, 1 - negi[d], negi[d])
        bits.append(bd)
        b = b + (bd << d)
    index2 = 112 + (b >> 1)

    idx = jnp.where(use1, index1, index2)

    y = []
    for d in range(8):
        sgn = jnp.where(neg[d], -1.0, 1.0)
        y_t1 = jnp.where((i1 == d) | (i2 == d), sgn, 0.0)
        y_t2 = jnp.where(bits[d] == 1, -0.5, 0.5)
        y.append(jnp.where(use1, y_t1, y_t2))
    return y, idx


def _body(x_ref, p_ref, pt_ref, q_ref, i1_ref, i2_ref, err_ref, xp_ref, qp_ref):
    xb = x_ref[...]
    xp_ref[...] = jax.lax.dot_general(
        xb, p_ref[...], (((1,), (0,)), ((), ())),
        preferred_element_type=jnp.float32,
        precision=jax.lax.Precision.HIGHEST,
    )

    err_acc = jnp.zeros((1, 1), jnp.float32)
    for s in range(_ROWS // _SUB):
        rows = pl.ds(s * _SUB, _SUB)
        xs = [xp_ref[rows, d * 128:(d + 1) * 128] for d in range(8)]
        y1, idx1 = _quantize_list([_bf16(v) for v in xs])
        r1 = [xs[d] - y1[d] for d in range(8)]
        y2, idx2 = _quantize_list([_bf16(v) for v in r1])
        qd = [y1[d] + y2[d] for d in range(8)]
        qp_ref[rows, :] = jnp.concatenate(qd, axis=1)
        i1_ref[0, rows, :] = idx1
        i2_ref[0, rows, :] = idx2
        e = jnp.zeros_like(err_acc)
        for d in range(8):
            rd = r1[d] - y2[d]
            e = e + jnp.sum(rd * rd).reshape(1, 1)
        err_acc = err_acc + e

    q_ref[...] = jax.lax.dot_general(
        qp_ref[...], pt_ref[...], (((1,), (0,)), ((), ())),
        preferred_element_type=jnp.float32,
    )

    @pl.when(pl.program_id(0) == 0)
    def _init():
        err_ref[...] = err_acc

    @pl.when(pl.program_id(0) != 0)
    def _acc():
        err_ref[...] += err_acc


@functools.partial(jax.jit, static_argnames=("interpret",))
def kernel(x, roots, interpret=False):
    del roots
    orig_shape = x.shape
    n = x.shape[0] * x.shape[1]
    nrows = n * 8 // 1024
    g = nrows // _ROWS

    xr = x.reshape(nrows, 1024)
    perm = jnp.asarray(_perm_np())
    permt = jnp.asarray(_perm_np().T.copy())

    q_r, idx1, idx2, err_sum = pl.pallas_call(
        _body,
        grid=(g,),
        in_specs=[
            pl.BlockSpec((_ROWS, 1024), lambda i: (i, 0)),
            pl.BlockSpec((1024, 1024), lambda i: (0, 0)),
            pl.BlockSpec((1024, 1024), lambda i: (0, 0)),
        ],
        out_specs=(
            pl.BlockSpec((_ROWS, 1024), lambda i: (i, 0)),
            pl.BlockSpec((1, _ROWS, 128), lambda i: (i, 0, 0)),
            pl.BlockSpec((1, _ROWS, 128), lambda i: (i, 0, 0)),
            pl.BlockSpec((1, 1), lambda i: (0, 0)),
        ),
        out_shape=(
            jax.ShapeDtypeStruct((nrows, 1024), jnp.float32),
            jax.ShapeDtypeStruct((g, _ROWS, 128), jnp.int32),
            jax.ShapeDtypeStruct((g, _ROWS, 128), jnp.int32),
            jax.ShapeDtypeStruct((1, 1), jnp.float32),
        ),
        scratch_shapes=[
            pltpu.VMEM((_ROWS, 1024), jnp.float32),
            pltpu.VMEM((_ROWS, 1024), jnp.float32),
        ],
        interpret=interpret,
    )(xr, perm, permt)

    quantized = q_r.reshape(orig_shape)
    i1 = idx1.reshape(orig_shape[:-1])
    i2 = idx2.reshape(orig_shape[:-1])
    err = err_sum[0, 0] / jnp.float32(n * 8)
    return (quantized, i1, i2, err)
